# baseline (speedup 1.0000x reference)
import jax
import jax.numpy as jnp
from jax.experimental import pallas as pl
from jax.experimental.pallas import tpu as pltpu

_BETA = 0.25


def _vq_body(h_ref, cb_ref, q_ref, g_ref, idx_ref, ss_ref, *, rb, k):
    h = h_ref[...]
    cb = cb_ref[...]
    h_sq = jnp.sum(h * h, axis=1, keepdims=True)
    c_sq = jnp.sum(cb * cb, axis=1).reshape(1, k)
    mm = jax.lax.dot_general(h, cb, (((1,), (1,)), ((), ())),
                             preferred_element_type=jnp.float32)
    dist = h_sq + c_sq - 2.0 * mm
    m = jnp.min(dist, axis=1, keepdims=True)
    iota = jax.lax.broadcasted_iota(jnp.int32, (rb, k), 1)
    idx = jnp.min(jnp.where(dist == m, iota, k), axis=1)
    q = (iota == idx[:, None]).astype(jnp.float32)
    q_ref[...] = q
    g = jax.lax.dot_general(q, cb, (((1,), (0,)), ((), ())),
                            preferred_element_type=jnp.float32)
    g_ref[...] = g
    idx_ref[...] = idx[:, None]
    ss_ref[...] = jnp.sum(m).reshape(1, 1, 1)


def _dual_body(htr_ref, hre_ref, cbtr_ref, cbre_ref,
               qtr_ref, gtr_ref, itr_ref, sstr_ref,
               qre_ref, gre_ref, ire_ref, ssre_ref, *, rb, k):
    _vq_body(htr_ref, cbtr_ref, qtr_ref, gtr_ref, itr_ref, sstr_ref, rb=rb, k=k)
    _vq_body(hre_ref, cbre_ref, qre_ref, gre_ref, ire_ref, ssre_ref, rb=rb, k=k)


def kernel(h_tr, h_re, codebook_tr, codebook_re):
    b, d = h_tr.shape
    k = codebook_tr.shape[0]
    rb = min(256, b)
    nb = b // rb

    import functools
    body = functools.partial(_dual_body, rb=rb, k=k)

    row_spec = pl.BlockSpec((rb, d), lambda i: (i, 0))
    cb_spec = pl.BlockSpec((k, d), lambda i: (0, 0))
    q_spec = pl.BlockSpec((rb, k), lambda i: (i, 0))
    idx_spec = pl.BlockSpec((rb, 1), lambda i: (i, 0))
    ss_spec = pl.BlockSpec((1, 1, 1), lambda i: (i, 0, 0))

    f32 = jnp.float32
    out_shapes = (
        jax.ShapeDtypeStruct((b, k), f32),
        jax.ShapeDtypeStruct((b, d), f32),
        jax.ShapeDtypeStruct((b, 1), jnp.int32),
        jax.ShapeDtypeStruct((nb, 1, 1), f32),
        jax.ShapeDtypeStruct((b, k), f32),
        jax.ShapeDtypeStruct((b, d), f32),
        jax.ShapeDtypeStruct((b, 1), jnp.int32),
        jax.ShapeDtypeStruct((nb, 1, 1), f32),
    )
    out_specs = (q_spec, row_spec, idx_spec, ss_spec,
                 q_spec, row_spec, idx_spec, ss_spec)

    (q_tr, g_tr, i_tr, ss_tr, q_re, g_re, i_re, ss_re) = pl.pallas_call(
        body,
        grid=(nb,),
        in_specs=[row_spec, row_spec, cb_spec, cb_spec],
        out_specs=out_specs,
        out_shape=out_shapes,
        compiler_params=pltpu.CompilerParams(
            dimension_semantics=("parallel",)),
    )(h_tr, h_re, codebook_tr, codebook_re)

    n = jnp.float32(b * d)
    total_loss = (1.0 + _BETA) * (jnp.sum(ss_tr) / n) + \
                 (1.0 + _BETA) * (jnp.sum(ss_re) / n)
    idx_tr = i_tr.reshape(b)
    idx_re = i_re.reshape(b)
    return (q_tr, g_tr, g_tr, g_tr, idx_tr,
            q_re, g_re, g_re, g_re, idx_re,
            total_loss)


# speedup vs baseline: 1.2114x; 1.2114x over previous
import jax
import jax.numpy as jnp
from jax.experimental import pallas as pl
from jax.experimental.pallas import tpu as pltpu

_BETA = 0.25


def _csq_body(cbtr_ref, cbre_ref, otr_ref, ore_ref, *, k):
    cbtr = cbtr_ref[...]
    otr_ref[...] = jnp.sum(cbtr * cbtr, axis=1).reshape(1, k)
    cbre = cbre_ref[...]
    ore_ref[...] = jnp.sum(cbre * cbre, axis=1).reshape(1, k)


def _vq_body(h_ref, cb_ref, csq_ref, q_ref, g_ref, idx_ref, ss_ref, *, rb, k):
    h = h_ref[...]
    cb = cb_ref[...]
    h_sq = jnp.sum(h * h, axis=1, keepdims=True)
    c_sq = csq_ref[...]
    mm = jax.lax.dot_general(h, cb, (((1,), (1,)), ((), ())),
                             preferred_element_type=jnp.float32)
    dist = h_sq + c_sq - 2.0 * mm
    idx = jnp.argmin(dist, axis=1)
    iota = jax.lax.broadcasted_iota(jnp.int32, (rb, k), 1)
    q = (iota == idx[:, None]).astype(jnp.float32)
    q_ref[...] = q
    g = jax.lax.dot_general(q, cb, (((1,), (0,)), ((), ())),
                            preferred_element_type=jnp.float32)
    g_ref[...] = g
    idx_ref[...] = idx[:, None]
    diff = h - g
    ss_ref[...] = jnp.sum(diff * diff).reshape(1, 1, 1)


def _dual_body(htr_ref, hre_ref, cbtr_ref, cbre_ref, csqtr_ref, csqre_ref,
               qtr_ref, gtr_ref, itr_ref, sstr_ref,
               qre_ref, gre_ref, ire_ref, ssre_ref, *, rb, k):
    _vq_body(htr_ref, cbtr_ref, csqtr_ref, qtr_ref, gtr_ref, itr_ref,
             sstr_ref, rb=rb, k=k)
    _vq_body(hre_ref, cbre_ref, csqre_ref, qre_ref, gre_ref, ire_ref,
             ssre_ref, rb=rb, k=k)


def kernel(h_tr, h_re, codebook_tr, codebook_re):
    b, d = h_tr.shape
    k = codebook_tr.shape[0]
    rb = min(256, b)
    nb = b // rb

    import functools
    body = functools.partial(_dual_body, rb=rb, k=k)

    f32 = jnp.float32
    csq_tr, csq_re = pl.pallas_call(
        functools.partial(_csq_body, k=k),
        out_shape=(jax.ShapeDtypeStruct((1, k), f32),
                   jax.ShapeDtypeStruct((1, k), f32)),
    )(codebook_tr, codebook_re)

    row_spec = pl.BlockSpec((rb, d), lambda i: (i, 0))
    cb_spec = pl.BlockSpec((k, d), lambda i: (0, 0))
    q_spec = pl.BlockSpec((rb, k), lambda i: (i, 0))
    idx_spec = pl.BlockSpec((rb, 1), lambda i: (i, 0))
    ss_spec = pl.BlockSpec((1, 1, 1), lambda i: (i, 0, 0))
    csq_spec = pl.BlockSpec((1, k), lambda i: (0, 0))

    out_shapes = (
        jax.ShapeDtypeStruct((b, k), f32),
        jax.ShapeDtypeStruct((b, d), f32),
        jax.ShapeDtypeStruct((b, 1), jnp.int32),
        jax.ShapeDtypeStruct((nb, 1, 1), f32),
        jax.ShapeDtypeStruct((b, k), f32),
        jax.ShapeDtypeStruct((b, d), f32),
        jax.ShapeDtypeStruct((b, 1), jnp.int32),
        jax.ShapeDtypeStruct((nb, 1, 1), f32),
    )
    out_specs = (q_spec, row_spec, idx_spec, ss_spec,
                 q_spec, row_spec, idx_spec, ss_spec)

    (q_tr, g_tr, i_tr, ss_tr, q_re, g_re, i_re, ss_re) = pl.pallas_call(
        body,
        grid=(nb,),
        in_specs=[row_spec, row_spec, cb_spec, cb_spec, csq_spec, csq_spec],
        out_specs=out_specs,
        out_shape=out_shapes,
        compiler_params=pltpu.CompilerParams(
            dimension_semantics=("parallel",)),
    )(h_tr, h_re, codebook_tr, codebook_re, csq_tr, csq_re)

    n = jnp.float32(b * d)
    total_loss = (1.0 + _BETA) * (jnp.sum(ss_tr) / n) + \
                 (1.0 + _BETA) * (jnp.sum(ss_re) / n)
    idx_tr = i_tr.reshape(b)
    idx_re = i_re.reshape(b)
    return (q_tr, g_tr, g_tr, g_tr, idx_tr,
            q_re, g_re, g_re, g_re, idx_re,
            total_loss)


# speedup vs baseline: 1.2159x; 1.0037x over previous
import functools

import jax
import jax.numpy as jnp
from jax import lax
from jax.experimental import pallas as pl
from jax.experimental.pallas import tpu as pltpu
from jax.experimental.pallas import tpu_sc as plsc

_BETA = 0.25


def _csq_body(cbtr_ref, cbre_ref, otr_ref, ore_ref, *, k):
    cbtr = cbtr_ref[...]
    otr_ref[...] = jnp.sum(cbtr * cbtr, axis=1).reshape(1, k)
    cbre = cbre_ref[...]
    ore_ref[...] = jnp.sum(cbre * cbre, axis=1).reshape(1, k)


def _vq_body(h_ref, cb_ref, csq_ref, q_ref, idx_ref, *, rb, k):
    h = h_ref[...]
    cb = cb_ref[...]
    h_sq = jnp.sum(h * h, axis=1, keepdims=True)
    c_sq = csq_ref[...]
    mm = jax.lax.dot_general(h, cb, (((1,), (1,)), ((), ())),
                             preferred_element_type=jnp.float32)
    dist = h_sq + c_sq - 2.0 * mm
    idx = jnp.argmin(dist, axis=1)
    iota = jax.lax.broadcasted_iota(jnp.int32, (rb, k), 1)
    q_ref[...] = (iota == idx[:, None]).astype(jnp.float32)
    idx_ref[...] = idx[:, None]


def _dual_body(htr_ref, hre_ref, cbtr_ref, cbre_ref, csqtr_ref, csqre_ref,
               qtr_ref, itr_ref, qre_ref, ire_ref, *, rb, k):
    _vq_body(htr_ref, cbtr_ref, csqtr_ref, qtr_ref, itr_ref, rb=rb, k=k)
    _vq_body(hre_ref, cbre_ref, csqre_ref, qre_ref, ire_ref, rb=rb, k=k)


def _sc_gather(cb_tr, idx_tr, cb_re, idx_re):
    info = plsc.get_sparse_core_info()
    nc, ns, l = info.num_cores, info.num_subcores, info.num_lanes
    nw = nc * ns
    b = idx_tr.shape[0]
    dp = cb_tr.shape[1]
    bw = b // nw
    f32 = jnp.float32
    mesh = plsc.VectorSubcoreMesh(core_axis_name="c", subcore_axis_name="s")

    @functools.partial(
        pl.kernel, mesh=mesh,
        out_type=(jax.ShapeDtypeStruct((b, dp), f32),
                  jax.ShapeDtypeStruct((b, dp), f32)),
        scratch_types=[
            pltpu.VMEM((bw,), jnp.int32),
            pltpu.VMEM((bw, dp), f32),
            pltpu.VMEM((bw,), jnp.int32),
            pltpu.VMEM((bw, dp), f32),
            pltpu.SemaphoreType.DMA,
            pltpu.SemaphoreType.DMA,
        ],
    )
    def k(cbtr_hbm, itr_hbm, cbre_hbm, ire_hbm,
          gtr_hbm, gre_hbm,
          itr_v, rtr_v, ire_v, rre_v, sem1, sem2):
        wid = lax.axis_index("s") * nc + lax.axis_index("c")
        base = wid * bw
        pltpu.sync_copy(itr_hbm.at[pl.ds(base, bw)], itr_v)
        cp1 = pltpu.async_copy(cbtr_hbm.at[itr_v], rtr_v, sem1)
        pltpu.sync_copy(ire_hbm.at[pl.ds(base, bw)], ire_v)
        cp2 = pltpu.async_copy(cbre_hbm.at[ire_v], rre_v, sem2)
        cp1.wait()
        pltpu.sync_copy(rtr_v, gtr_hbm.at[pl.ds(base, bw)])
        cp2.wait()
        pltpu.sync_copy(rre_v, gre_hbm.at[pl.ds(base, bw)])

    return k(cb_tr, idx_tr, cb_re, idx_re)


def _loss_body(htr_ref, gtr_ref, hre_ref, gre_ref, str_ref, sre_ref):
    dtr = htr_ref[...] - gtr_ref[...]
    str_ref[...] = jnp.sum(dtr * dtr).reshape(1, 1)
    dre = hre_ref[...] - gre_ref[...]
    sre_ref[...] = jnp.sum(dre * dre).reshape(1, 1)


def kernel(h_tr, h_re, codebook_tr, codebook_re):
    b, d = h_tr.shape
    k = codebook_tr.shape[0]
    rb = min(256, b)
    nb = b // rb

    body = functools.partial(_dual_body, rb=rb, k=k)

    f32 = jnp.float32
    csq_tr, csq_re = pl.pallas_call(
        functools.partial(_csq_body, k=k),
        out_shape=(jax.ShapeDtypeStruct((1, k), f32),
                   jax.ShapeDtypeStruct((1, k), f32)),
    )(codebook_tr, codebook_re)

    row_spec = pl.BlockSpec((rb, d), lambda i: (i, 0))
    cb_spec = pl.BlockSpec((k, d), lambda i: (0, 0))
    q_spec = pl.BlockSpec((rb, k), lambda i: (i, 0))
    idx_spec = pl.BlockSpec((rb, 1), lambda i: (i, 0))
    csq_spec = pl.BlockSpec((1, k), lambda i: (0, 0))

    out_shapes = (
        jax.ShapeDtypeStruct((b, k), f32),
        jax.ShapeDtypeStruct((b, 1), jnp.int32),
        jax.ShapeDtypeStruct((b, k), f32),
        jax.ShapeDtypeStruct((b, 1), jnp.int32),
    )
    out_specs = (q_spec, idx_spec, q_spec, idx_spec)

    (q_tr, i_tr, q_re, i_re) = pl.pallas_call(
        body,
        grid=(nb,),
        in_specs=[row_spec, row_spec, cb_spec, cb_spec, csq_spec, csq_spec],
        out_specs=out_specs,
        out_shape=out_shapes,
        compiler_params=pltpu.CompilerParams(
            dimension_semantics=("parallel",)),
    )(h_tr, h_re, codebook_tr, codebook_re, csq_tr, csq_re)

    idx_tr = i_tr.reshape(b)
    idx_re = i_re.reshape(b)

    cb_tr_pad = jnp.pad(codebook_tr, ((0, 0), (0, 128 - d)))
    cb_re_pad = jnp.pad(codebook_re, ((0, 0), (0, 128 - d)))
    g_tr_pad, g_re_pad = _sc_gather(cb_tr_pad, idx_tr, cb_re_pad, idx_re)
    g_tr = g_tr_pad[:, :d]
    g_re = g_re_pad[:, :d]

    ss_tr, ss_re = pl.pallas_call(
        _loss_body,
        out_shape=(jax.ShapeDtypeStruct((1, 1), f32),
                   jax.ShapeDtypeStruct((1, 1), f32)),
    )(h_tr, g_tr, h_re, g_re)

    n = jnp.float32(b * d)
    total_loss = (1.0 + _BETA) * (ss_tr[0, 0] / n) + \
                 (1.0 + _BETA) * (ss_re[0, 0] / n)
    return (q_tr, g_tr, g_tr, g_tr, idx_tr,
            q_re, g_re, g_re, g_re, idx_re,
            total_loss)
